# baseline (speedup 1.0000x reference)
import functools

import jax
import jax.numpy as jnp
from jax import lax
from jax.experimental import pallas as pl
from jax.experimental.pallas import tpu as pltpu
from jax.experimental.pallas import tpu_sc as plsc

_B = 16384
_PAD = 50
_D = 64
_TOTAL = _B * _PAD
_NC, _NS = 2, 16
_NW = _NC * _NS
_PER_W = _TOTAL // _NW
_C = 128
_G = 4
_ROWS_G = _G * _C
_NGROUP = _PER_W // _ROWS_G
_T = _NGROUP // 2


def _make_emb():
    mesh = plsc.VectorSubcoreMesh(core_axis_name="c", subcore_axis_name="s")

    @functools.partial(
        pl.kernel,
        mesh=mesh,
        out_type=jax.ShapeDtypeStruct((_TOTAL, _D), jnp.float32),
        scratch_types=[
            pltpu.VMEM((2, _G, _C), jnp.int32),
            pltpu.VMEM((2, _ROWS_G, _D), jnp.float32),
            pltpu.SemaphoreType.DMA,
            pltpu.SemaphoreType.DMA,
            pltpu.SemaphoreType.DMA,
            pltpu.SemaphoreType.DMA,
            pltpu.SemaphoreType.DMA,
            pltpu.SemaphoreType.DMA,
        ],
        compiler_params=pltpu.CompilerParams(use_tc_tiling_on_sc=False),
    )
    def emb(idx_hbm, table_hbm, out_hbm, idx_v, rows_v,
            isem0, isem1, gsem0, gsem1, osem0, osem1):
        wid = lax.axis_index("s") * _NC + lax.axis_index("c")
        cbase = wid * (_PER_W // _C)
        rbase = wid * _PER_W
        isems = (isem0, isem1)
        gsems = (gsem0, gsem1)
        osems = (osem0, osem1)

        def idx_cp(g, b):
            return pltpu.make_async_copy(
                idx_hbm.at[pl.ds(cbase + g * _G, _G)], idx_v.at[b], isems[b])

        def gather_cp(b, j):
            return pltpu.make_async_copy(
                table_hbm.at[idx_v.at[b, j]],
                rows_v.at[b, pl.ds(j * _C, _C)], gsems[b])

        def out_cp(g, b):
            return pltpu.make_async_copy(
                rows_v.at[b], out_hbm.at[pl.ds(rbase + g * _ROWS_G, _ROWS_G)],
                osems[b])

        idx_cp(0, 0).start()
        idx_cp(1, 1).start()

        def body(t, carry):
            for b in range(2):
                g = 2 * t + b
                idx_cp(g, b).wait()

                @pl.when(t >= 1)
                def _():
                    out_cp(g - 2, b).wait()

                for j in range(_G):
                    gather_cp(b, j).start()
                for j in range(_G):
                    gather_cp(b, j).wait()

                out_cp(g, b).start()

                @pl.when(t < _T - 1)
                def _():
                    idx_cp(g + 2, b).start()
            return carry

        lax.fori_loop(0, _T, body, 0)

        out_cp(_NGROUP - 2, 0).wait()
        out_cp(_NGROUP - 1, 1).wait()

    return emb


_emb = _make_emb()


@jax.jit
def kernel(indices, emb_weight):
    idx2d = indices.reshape(_TOTAL // _C, _C)
    out = _emb(idx2d, emb_weight)
    return out.reshape(_B, _PAD, _D)


# speedup vs baseline: 2.7116x; 2.7116x over previous
import functools

import jax
import jax.numpy as jnp
from jax import lax
from jax.experimental import pallas as pl
from jax.experimental.pallas import tpu as pltpu
from jax.experimental.pallas import tpu_sc as plsc

_B = 16384
_PAD = 50
_D = 64
_TOTAL = _B * _PAD
_NC, _NS = 2, 16
_NW = _NC * _NS
_PER_W = _TOTAL // _NW
_C = 128
_G = 4
_ROWS_G = _G * _C
_NGROUP = _PER_W // _ROWS_G
_T = _NGROUP // 2


def _make_emb():
    mesh = plsc.VectorSubcoreMesh(core_axis_name="c", subcore_axis_name="s")

    @functools.partial(
        pl.kernel,
        mesh=mesh,
        out_type=jax.ShapeDtypeStruct((_TOTAL, _D), jnp.float32),
        scratch_types=[
            pltpu.VMEM_SHARED((60, _D), jnp.float32),
            pltpu.VMEM((2, _G, _C), jnp.int32),
            pltpu.VMEM((2, _ROWS_G, _D), jnp.float32),
            pltpu.SemaphoreType.DMA,
            pltpu.SemaphoreType.DMA,
            pltpu.SemaphoreType.DMA,
            pltpu.SemaphoreType.DMA,
            pltpu.SemaphoreType.DMA,
            pltpu.SemaphoreType.DMA,
        ],
        compiler_params=pltpu.CompilerParams(use_tc_tiling_on_sc=False),
    )
    def emb(idx_hbm, table_hbm, out_hbm, table_v, idx_v, rows_v,
            isem0, isem1, gsem0, gsem1, osem0, osem1):
        wid = lax.axis_index("s") * _NC + lax.axis_index("c")
        cbase = wid * (_PER_W // _C)
        rbase = wid * _PER_W
        isems = (isem0, isem1)
        gsems = (gsem0, gsem1)
        osems = (osem0, osem1)

        def idx_cp(g, b):
            return pltpu.make_async_copy(
                idx_hbm.at[pl.ds(cbase + g * _G, _G)], idx_v.at[b], isems[b])

        def gather_cp(b, j):
            return pltpu.make_async_copy(
                table_v.at[idx_v.at[b, j]],
                rows_v.at[b, pl.ds(j * _C, _C)], gsems[b])

        def out_cp(g, b):
            return pltpu.make_async_copy(
                rows_v.at[b], out_hbm.at[pl.ds(rbase + g * _ROWS_G, _ROWS_G)],
                osems[b])

        @pl.when(lax.axis_index("s") == 0)
        def _():
            pltpu.sync_copy(table_hbm, table_v)
        plsc.subcore_barrier()

        idx_cp(0, 0).start()
        idx_cp(1, 1).start()

        def body(t, carry):
            for b in range(2):
                g = 2 * t + b
                idx_cp(g, b).wait()

                @pl.when(t >= 1)
                def _():
                    out_cp(g - 2, b).wait()

                for j in range(_G):
                    gather_cp(b, j).start()
                for j in range(_G):
                    gather_cp(b, j).wait()

                out_cp(g, b).start()

                @pl.when(t < _T - 1)
                def _():
                    idx_cp(g + 2, b).start()
            return carry

        lax.fori_loop(0, _T, body, 0)

        out_cp(_NGROUP - 2, 0).wait()
        out_cp(_NGROUP - 1, 1).wait()

    return emb


_emb = _make_emb()


@jax.jit
def kernel(indices, emb_weight):
    idx2d = indices.reshape(_TOTAL // _C, _C)
    out = _emb(idx2d, emb_weight)
    return out.reshape(_B, _PAD, _D)
